# baseline (speedup 1.0000x reference)
import functools

import jax
import jax.numpy as jnp
from jax import lax
from jax.experimental import pallas as pl
from jax.experimental.pallas import tpu as pltpu
from jax.experimental.pallas import tpu_sc as plsc

EMBED_DIM = 64
BLK = 128
LANES = 16


def _make_gather(n_sent: int, seq: int, nw: int):
    n_tiles = seq * BLK
    nbh = n_sent // BLK
    n_t = seq * nbh // nw
    assert (seq * nbh) % nw == 0 and n_t % 2 == 0

    mesh = plsc.VectorSubcoreMesh(core_axis_name="c", subcore_axis_name="s")

    @functools.partial(
        pl.kernel,
        mesh=mesh,
        out_type=jax.ShapeDtypeStruct((seq, EMBED_DIM // 8, nbh, 8, BLK),
                                      jnp.float32),
        scratch_types=[
            pltpu.VMEM((2, BLK), jnp.int32),
            pltpu.VMEM((2, BLK, EMBED_DIM), jnp.float32),
            pltpu.VMEM((2, EMBED_DIM // 8, 8, BLK), jnp.float32),
            pltpu.SemaphoreType.DMA,
            pltpu.SemaphoreType.DMA,
            pltpu.SemaphoreType.DMA,
            pltpu.SemaphoreType.DMA,
        ],
        compiler_params=pltpu.CompilerParams(use_tc_tiling_on_sc=False,
                                             needs_layout_passes=False),
    )
    def k(table_hbm, idx_hbm, out_hbm, idx_v, rowbuf, tbuf,
          sg0, sg1, so0, so1):
        nc = 2
        wid = lax.axis_index("s") * nc + lax.axis_index("c")
        t0 = wid * n_t
        sg = (sg0, sg1)
        so = (so0, so1)
        iota = lax.iota(jnp.int32, LANES)

        def fire(t, slot):
            s1 = t // nbh
            bh1 = t % nbh
            pltpu.sync_copy(idx_hbm.at[s1, pl.ds(bh1 * BLK, BLK)],
                            idx_v.at[slot])
            pltpu.async_copy(table_hbm.at[idx_v.at[slot]],
                             rowbuf.at[slot], sg[slot])

        def drain(slot):
            pltpu.make_async_copy(table_hbm.at[idx_v.at[slot]],
                                  rowbuf.at[slot], sg[slot]).wait()

        def transpose(slot):
            src = rowbuf.at[slot]

            def tr_body(g, _):
                b_vec = g * LANES + iota
                for e in range(EMBED_DIM):
                    e_vec = jnp.full((LANES,), e, dtype=jnp.int32)
                    vec = plsc.load_gather(src, [b_vec, e_vec])
                    tbuf[slot, e // 8, e % 8, pl.ds(g * LANES, LANES)] = vec
                return 0

            lax.fori_loop(0, BLK // LANES, tr_body, 0)

        def write_out(t, slot):
            s1 = t // nbh
            bh1 = t % nbh
            pltpu.async_copy(tbuf.at[slot],
                             out_hbm.at[s1, :, bh1], so[slot])

        def wait_write(t, slot):
            s1 = t // nbh
            bh1 = t % nbh
            pltpu.make_async_copy(tbuf.at[slot],
                                  out_hbm.at[s1, :, bh1], so[slot]).wait()

        fire(t0, 0)

        def body(i, _):
            for slot in (0, 1):
                t = t0 + 2 * i + slot
                other = 1 - slot
                if slot == 0:
                    @pl.when(i >= 1)
                    def _():
                        wait_write(t - 2, slot)

                    fire(t + 1, other)
                else:
                    @pl.when(i >= 1)
                    def _():
                        wait_write(t - 2, slot)

                    @pl.when(2 * i + slot + 1 < n_t)
                    def _():
                        fire(t + 1, other)
                drain(slot)
                transpose(slot)
                write_out(t, slot)
            return 0

        lax.fori_loop(0, n_t // 2, body, 0)
        wait_write(t0 + n_t - 2, 0)
        wait_write(t0 + n_t - 1, 1)

    return k


def kernel(table, input):
    n_sent, seq = input.shape
    idx_t = input.T.astype(jnp.int32)
    p5 = _make_gather(n_sent, seq, 32)(table, idx_t)
    return p5.transpose(2, 4, 0, 1, 3).reshape(n_sent, seq, EMBED_DIM)


# speedup vs baseline: 2.1847x; 2.1847x over previous
import functools

import jax
import jax.numpy as jnp
from jax import lax
from jax.experimental import pallas as pl
from jax.experimental.pallas import tpu as pltpu
from jax.experimental.pallas import tpu_sc as plsc

EMBED_DIM = 64
BLK = 128
LANES = 16


def _make_gather(n_sent: int, seq: int, nw: int):
    n_tiles = seq * BLK
    nbh = n_sent // BLK
    n_t = seq * nbh // nw
    assert (seq * nbh) % nw == 0 and n_t % 2 == 0

    mesh = plsc.VectorSubcoreMesh(core_axis_name="c", subcore_axis_name="s")

    @functools.partial(
        pl.kernel,
        mesh=mesh,
        out_type=jax.ShapeDtypeStruct((seq, EMBED_DIM // 8, nbh, 8, BLK),
                                      jnp.float32),
        scratch_types=[
            pltpu.VMEM((2, BLK), jnp.int32),
            pltpu.VMEM((2, BLK, EMBED_DIM), jnp.float32),
            pltpu.VMEM((2, EMBED_DIM // 8, 8, BLK + 1), jnp.float32),
            pltpu.SemaphoreType.DMA,
            pltpu.SemaphoreType.DMA,
            pltpu.SemaphoreType.DMA,
            pltpu.SemaphoreType.DMA,
        ],
        compiler_params=pltpu.CompilerParams(use_tc_tiling_on_sc=False,
                                             needs_layout_passes=False),
    )
    def k(table_hbm, idx_hbm, out_hbm, idx_v, rowbuf, tbuf,
          sg0, sg1, so0, so1):
        nc = 2
        wid = lax.axis_index("s") * nc + lax.axis_index("c")
        t0 = wid * n_t
        sg = (sg0, sg1)
        so = (so0, so1)
        iota = lax.iota(jnp.int32, LANES)

        def fire(t, slot):
            s1 = t // nbh
            bh1 = t % nbh
            pltpu.sync_copy(idx_hbm.at[s1, pl.ds(bh1 * BLK, BLK)],
                            idx_v.at[slot])
            pltpu.async_copy(table_hbm.at[idx_v.at[slot]],
                             rowbuf.at[slot], sg[slot])

        def drain(slot):
            pltpu.make_async_copy(table_hbm.at[idx_v.at[slot]],
                                  rowbuf.at[slot], sg[slot]).wait()

        eh_vecs = [(iota + e0) // 8 for e0 in range(0, EMBED_DIM, LANES)]
        el_vecs = [(iota + e0) % 8 for e0 in range(0, EMBED_DIM, LANES)]

        def transpose(slot):
            dst = tbuf.at[slot]

            def tr_body(b, _):
                b_vec = jnp.full((LANES,), 0, dtype=jnp.int32) + b
                for j in range(EMBED_DIM // LANES):
                    vec = rowbuf[slot, b, pl.ds(j * LANES, LANES)]
                    plsc.store_scatter(dst, [eh_vecs[j], el_vecs[j], b_vec],
                                       vec)
                return 0

            lax.fori_loop(0, BLK, tr_body, 0)

        def write_out(t, slot):
            s1 = t // nbh
            bh1 = t % nbh
            pltpu.async_copy(tbuf.at[slot, :, :, pl.ds(0, BLK)],
                             out_hbm.at[s1, :, bh1], so[slot])

        def wait_write(t, slot):
            s1 = t // nbh
            bh1 = t % nbh
            pltpu.make_async_copy(tbuf.at[slot, :, :, pl.ds(0, BLK)],
                                  out_hbm.at[s1, :, bh1], so[slot]).wait()

        fire(t0, 0)

        def body(i, _):
            for slot in (0, 1):
                t = t0 + 2 * i + slot
                other = 1 - slot
                if slot == 0:
                    @pl.when(i >= 1)
                    def _():
                        wait_write(t - 2, slot)

                    fire(t + 1, other)
                else:
                    @pl.when(i >= 1)
                    def _():
                        wait_write(t - 2, slot)

                    @pl.when(2 * i + slot + 1 < n_t)
                    def _():
                        fire(t + 1, other)
                drain(slot)
                transpose(slot)
                write_out(t, slot)
            return 0

        lax.fori_loop(0, n_t // 2, body, 0)
        wait_write(t0 + n_t - 2, 0)
        wait_write(t0 + n_t - 1, 1)

    return k


def kernel(table, input):
    n_sent, seq = input.shape
    idx_t = input.T.astype(jnp.int32)
    p5 = _make_gather(n_sent, seq, 32)(table, idx_t)
    return p5.transpose(2, 4, 0, 1, 3).reshape(n_sent, seq, EMBED_DIM)


# speedup vs baseline: 2.8157x; 1.2888x over previous
import functools

import jax
import jax.numpy as jnp
from jax import lax
from jax.experimental import pallas as pl
from jax.experimental.pallas import tpu as pltpu
from jax.experimental.pallas import tpu_sc as plsc

EMBED_DIM = 64
BLK = 128
LANES = 16


def _make_gather(n_sent: int, seq: int, nw: int):
    n_tiles = seq * BLK
    nbh = n_sent // BLK
    n_t = seq * nbh // nw
    assert (seq * nbh) % nw == 0 and n_t % 2 == 0

    mesh = plsc.VectorSubcoreMesh(core_axis_name="c", subcore_axis_name="s")

    @functools.partial(
        pl.kernel,
        mesh=mesh,
        out_type=jax.ShapeDtypeStruct((seq, EMBED_DIM // 8, nbh, 8, BLK),
                                      jnp.float32),
        scratch_types=[
            pltpu.VMEM((2, BLK), jnp.int32),
            pltpu.VMEM((2, BLK, EMBED_DIM), jnp.float32),
            pltpu.VMEM((2, EMBED_DIM // 8, 8, BLK + 1), jnp.float32),
            pltpu.SemaphoreType.DMA,
            pltpu.SemaphoreType.DMA,
            pltpu.SemaphoreType.DMA,
            pltpu.SemaphoreType.DMA,
        ],
        compiler_params=pltpu.CompilerParams(use_tc_tiling_on_sc=False,
                                             needs_layout_passes=False),
    )
    def k(table_hbm, idx_hbm, out_hbm, idx_v, rowbuf, tbuf,
          sg0, sg1, so0, so1):
        nc = 2
        wid = lax.axis_index("s") * nc + lax.axis_index("c")
        t0 = wid * n_t
        sg = (sg0, sg1)
        so = (so0, so1)
        iota = lax.iota(jnp.int32, LANES)

        def fire(t, slot):
            s1 = t // nbh
            bh1 = t % nbh
            pltpu.sync_copy(idx_hbm.at[s1, pl.ds(bh1 * BLK, BLK)],
                            idx_v.at[slot])
            pltpu.async_copy(table_hbm.at[idx_v.at[slot]],
                             rowbuf.at[slot], sg[slot])

        def drain(slot):
            pltpu.make_async_copy(table_hbm.at[idx_v.at[slot]],
                                  rowbuf.at[slot], sg[slot]).wait()

        eh_vecs = [(iota + e0) // 8 for e0 in range(0, EMBED_DIM, LANES)]
        el_vecs = [(iota + e0) % 8 for e0 in range(0, EMBED_DIM, LANES)]

        def transpose(slot):
            dst = tbuf.at[slot]
            nj = EMBED_DIM // LANES

            def loads(b):
                return tuple(rowbuf[slot, b, pl.ds(j * LANES, LANES)]
                             for j in range(nj))

            def tr_body(b, vecs):
                nxt = loads(b + 1)
                b_vec = jnp.full((LANES,), 0, dtype=jnp.int32) + b
                for j in range(nj):
                    plsc.store_scatter(dst, [eh_vecs[j], el_vecs[j], b_vec],
                                       vecs[j])
                return nxt

            last = lax.fori_loop(0, BLK - 1, tr_body, loads(0))
            b_vec = jnp.full((LANES,), BLK - 1, dtype=jnp.int32)
            for j in range(nj):
                plsc.store_scatter(dst, [eh_vecs[j], el_vecs[j], b_vec],
                                   last[j])

        def write_out(t, slot):
            s1 = t // nbh
            bh1 = t % nbh
            pltpu.async_copy(tbuf.at[slot, :, :, pl.ds(0, BLK)],
                             out_hbm.at[s1, :, bh1], so[slot])

        def wait_write(t, slot):
            s1 = t // nbh
            bh1 = t % nbh
            pltpu.make_async_copy(tbuf.at[slot, :, :, pl.ds(0, BLK)],
                                  out_hbm.at[s1, :, bh1], so[slot]).wait()

        fire(t0, 0)

        def body(i, _):
            for slot in (0, 1):
                t = t0 + 2 * i + slot
                other = 1 - slot
                if slot == 0:
                    @pl.when(i >= 1)
                    def _():
                        wait_write(t - 2, slot)

                    fire(t + 1, other)
                else:
                    @pl.when(i >= 1)
                    def _():
                        wait_write(t - 2, slot)

                    @pl.when(2 * i + slot + 1 < n_t)
                    def _():
                        fire(t + 1, other)
                drain(slot)
                transpose(slot)
                write_out(t, slot)
            return 0

        lax.fori_loop(0, n_t // 2, body, 0)
        wait_write(t0 + n_t - 2, 0)
        wait_write(t0 + n_t - 1, 1)

    return k


def kernel(table, input):
    n_sent, seq = input.shape
    idx_t = input.T.astype(jnp.int32)
    p5 = _make_gather(n_sent, seq, 32)(table, idx_t)
    return p5.transpose(2, 4, 0, 1, 3).reshape(n_sent, seq, EMBED_DIM)


# speedup vs baseline: 3.6276x; 1.2883x over previous
import functools

import jax
import jax.numpy as jnp
from jax import lax
from jax.experimental import pallas as pl
from jax.experimental.pallas import tpu as pltpu
from jax.experimental.pallas import tpu_sc as plsc

EMBED_DIM = 64
BLK = 128
LANES = 16
IBATCH = 16


def _make_gather(n_sent: int, seq: int, nw: int):
    nbh = n_sent // BLK
    n_t = seq * nbh // nw
    assert (seq * nbh) % nw == 0 and n_t % 2 == 0 and n_t % IBATCH == 0
    assert nbh % IBATCH == 0

    mesh = plsc.VectorSubcoreMesh(core_axis_name="c", subcore_axis_name="s")

    @functools.partial(
        pl.kernel,
        mesh=mesh,
        out_type=jax.ShapeDtypeStruct((seq, EMBED_DIM // 8, nbh, 8, BLK),
                                      jnp.float32),
        scratch_types=[
            pltpu.VMEM((2, IBATCH, BLK), jnp.int32),
            pltpu.VMEM((2, BLK, EMBED_DIM), jnp.float32),
            pltpu.VMEM((2, EMBED_DIM // 8, 8, BLK + 1), jnp.float32),
            pltpu.SemaphoreType.DMA,
            pltpu.SemaphoreType.DMA,
            pltpu.SemaphoreType.DMA,
            pltpu.SemaphoreType.DMA,
            pltpu.SemaphoreType.DMA,
            pltpu.SemaphoreType.DMA,
        ],
        compiler_params=pltpu.CompilerParams(use_tc_tiling_on_sc=False,
                                             needs_layout_passes=False),
    )
    def k(table_hbm, idx_hbm, out_hbm, idxbuf, rowbuf, tbuf,
          sg0, sg1, so0, so1, si0, si1):
        nc = 2
        wid = lax.axis_index("s") * nc + lax.axis_index("c")
        t0 = wid * n_t
        sg = (sg0, sg1)
        so = (so0, so1)
        si = (si0, si1)
        iota = lax.iota(jnp.int32, LANES)

        eh_vecs = [(iota + e0) // 8 for e0 in range(0, EMBED_DIM, LANES)]
        el_vecs = [(iota + e0) % 8 for e0 in range(0, EMBED_DIM, LANES)]

        def idx_copy(kb, slot):
            tt = t0 + kb * IBATCH
            return pltpu.make_async_copy(
                idx_hbm.at[tt // nbh, pl.ds(tt % nbh, IBATCH)],
                idxbuf.at[slot], si[slot])

        def idx_load(kb, slot):
            tt = t0 + kb * IBATCH
            pltpu.async_copy(idx_hbm.at[tt // nbh, pl.ds(tt % nbh, IBATCH)],
                             idxbuf.at[slot], si[slot])

        def fire(t, slot):
            rel = t - t0
            gslot = (rel // IBATCH) % 2
            pltpu.async_copy(table_hbm.at[idxbuf.at[gslot, rel % IBATCH]],
                             rowbuf.at[slot], sg[slot])

        def drain(t, slot):
            rel = t - t0
            gslot = (rel // IBATCH) % 2
            pltpu.make_async_copy(table_hbm.at[idxbuf.at[gslot, rel % IBATCH]],
                                  rowbuf.at[slot], sg[slot]).wait()

        def prefetch(t):
            rel = t - t0

            kb_l = rel // IBATCH + 1
            cond_l = (rel % IBATCH == 0) & (rel + IBATCH < n_t)
            kb_w = (rel + 1) // IBATCH
            cond_w = ((rel + 1) % IBATCH == 0) & (rel + 1 < n_t)
            for par in (0, 1):
                @pl.when(cond_l & (kb_l % 2 == par))
                def _(kb=kb_l, par=par):
                    idx_load(kb, par)

                @pl.when(cond_w & (kb_w % 2 == par))
                def _(kb=kb_w, par=par):
                    idx_copy(kb, par).wait()

        def transpose(slot):
            dst = tbuf.at[slot]
            nj = EMBED_DIM // LANES

            def loads(b):
                return tuple(rowbuf[slot, b, pl.ds(j * LANES, LANES)]
                             for j in range(nj))

            def tr_body(b, vecs):
                nxt = loads(b + 1)
                b_vec = jnp.full((LANES,), 0, dtype=jnp.int32) + b
                for j in range(nj):
                    plsc.store_scatter(dst, [eh_vecs[j], el_vecs[j], b_vec],
                                       vecs[j])
                return nxt

            last = lax.fori_loop(0, BLK - 1, tr_body, loads(0))
            b_vec = jnp.full((LANES,), BLK - 1, dtype=jnp.int32)
            for j in range(nj):
                plsc.store_scatter(dst, [eh_vecs[j], el_vecs[j], b_vec],
                                   last[j])

        def write_out(t, slot):
            pltpu.async_copy(tbuf.at[slot, :, :, pl.ds(0, BLK)],
                             out_hbm.at[t // nbh, :, t % nbh], so[slot])

        def wait_write(t, slot):
            pltpu.make_async_copy(tbuf.at[slot, :, :, pl.ds(0, BLK)],
                                  out_hbm.at[t // nbh, :, t % nbh],
                                  so[slot]).wait()

        idx_load(0, 0)
        idx_copy(0, 0).wait()
        fire(t0, 0)

        def body(i, _):
            for slot in (0, 1):
                t = t0 + 2 * i + slot
                other = 1 - slot

                @pl.when(i >= 1)
                def _():
                    wait_write(t - 2, slot)

                prefetch(t)
                if slot == 0:
                    fire(t + 1, other)
                else:
                    @pl.when(2 * i + slot + 1 < n_t)
                    def _():
                        fire(t + 1, other)
                drain(t, slot)
                transpose(slot)
                write_out(t, slot)
            return 0

        lax.fori_loop(0, n_t // 2, body, 0)
        wait_write(t0 + n_t - 2, 0)
        wait_write(t0 + n_t - 1, 1)

    return k


def kernel(table, input):
    n_sent, seq = input.shape
    idx_t = input.T.astype(jnp.int32).reshape(seq, n_sent // BLK, BLK)
    p5 = _make_gather(n_sent, seq, 32)(table, idx_t)
    return p5.transpose(2, 4, 0, 1, 3).reshape(n_sent, seq, EMBED_DIM)
